# Initial kernel scaffold; baseline (speedup 1.0000x reference)
import jax
import jax.numpy as jnp
from jax.experimental import pallas as pl


def kernel(x, edge_index, cutoff, W1, b1, gamma, beta, W2, b2, Wl, bl):
    raise NotImplementedError("write your pallas kernel here")


# speedup vs baseline: 40.2821x; 40.2821x over previous
import functools

import jax
import jax.numpy as jnp
from jax import lax
from jax.experimental import pallas as pl
from jax.experimental.pallas import tpu as pltpu
from jax.experimental.pallas import tpu_sc as plsc

E = 320000
V = 10000
V2 = 10240
D = 128

NC, NS = 2, 16
NW = NC * NS
EC = E // NW
K = 80
NCHUNK = EC // K
VROWS = V2 // NS

BR = 2000
HEAD_BLKS = V // BR
TAIL_BLKS = (E - V) // BR

_MESH = plsc.VectorSubcoreMesh(core_axis_name="c", subcore_axis_name="s")


def _sc_hist(dst, zeros16, ones16):
    @functools.partial(
        pl.kernel,
        out_type=jax.ShapeDtypeStruct((NC, V2, D), jnp.float32),
        mesh=_MESH,
        scratch_types=[
            pltpu.VMEM((K,), jnp.int32),
            pltpu.VMEM((K, D), jnp.float32),
            pltpu.VMEM_SHARED((V2, D), jnp.float32),
        ],
    )
    def hist(dst_hbm, z_hbm, ones_hbm, out_hbm, didx, ones_v, acc_sh):
        c = lax.axis_index("c")
        s = lax.axis_index("s")
        w = c * NS + s
        pltpu.sync_copy(z_hbm.at[pl.ds(s * VROWS, VROWS)],
                        acc_sh.at[pl.ds(s * VROWS, VROWS)])
        pltpu.sync_copy(ones_hbm, ones_v)
        plsc.subcore_barrier()

        def chunk(i, carry):
            base = w * EC + i * K
            pltpu.sync_copy(dst_hbm.at[pl.ds(base, K)], didx)
            pltpu.sync_copy(ones_v, acc_sh.at[didx], add=True)
            return carry

        lax.fori_loop(0, NCHUNK, chunk, 0)
        plsc.subcore_barrier()
        pltpu.sync_copy(acc_sh.at[pl.ds(s * VROWS, VROWS)],
                        out_hbm.at[c, pl.ds(s * VROWS, VROWS)])

    return hist(dst, zeros16, ones16)


def _sc_gather_add(src, dst, A, B):
    @functools.partial(
        pl.kernel,
        out_type=jax.ShapeDtypeStruct((E, D), jnp.float32),
        mesh=_MESH,
        scratch_types=[
            pltpu.VMEM((K,), jnp.int32),
            pltpu.VMEM((K,), jnp.int32),
            pltpu.VMEM((K, D), jnp.float32),
            pltpu.VMEM((K, D), jnp.float32),
            pltpu.SemaphoreType.DMA,
            pltpu.SemaphoreType.DMA,
        ],
    )
    def gather(src_hbm, dst_hbm, a_hbm, b_hbm, out_hbm,
               sidx, didx, bufa, bufb, sema, semb):
        c = lax.axis_index("c")
        s = lax.axis_index("s")
        w = c * NS + s

        def chunk(i, carry):
            base = w * EC + i * K
            pltpu.sync_copy(src_hbm.at[pl.ds(base, K)], sidx)
            pltpu.sync_copy(dst_hbm.at[pl.ds(base, K)], didx)
            cpa = pltpu.async_copy(a_hbm.at[sidx], bufa, sema)
            cpb = pltpu.async_copy(b_hbm.at[didx], bufb, semb)
            cpa.wait()
            cpb.wait()

            def row(r, rc):
                for j in range(D // 16):
                    sl = pl.ds(j * 16, 16)
                    bufa[r, sl] = bufa[r, sl] + bufb[r, sl]
                return rc

            lax.fori_loop(0, K, row, 0)
            pltpu.sync_copy(bufa, out_hbm.at[pl.ds(base, K)])
            return carry

        lax.fori_loop(0, NCHUNK, chunk, 0)

    return gather(src, dst, A, B)


def _sc_scatter(src, dst, table, zerosVD):
    @functools.partial(
        pl.kernel,
        out_type=jax.ShapeDtypeStruct((NC, V2, D), jnp.float32),
        mesh=_MESH,
        scratch_types=[
            pltpu.VMEM((K,), jnp.int32),
            pltpu.VMEM((K,), jnp.int32),
            pltpu.VMEM((K, D), jnp.float32),
            pltpu.VMEM_SHARED((V2, D), jnp.float32),
            pltpu.SemaphoreType.DMA,
        ],
    )
    def scat(src_hbm, dst_hbm, t_hbm, z_hbm, out_hbm,
             sidx, didx, buf, acc_sh, sem):
        c = lax.axis_index("c")
        s = lax.axis_index("s")
        w = c * NS + s
        pltpu.sync_copy(z_hbm.at[pl.ds(s * VROWS, VROWS)],
                        acc_sh.at[pl.ds(s * VROWS, VROWS)])
        plsc.subcore_barrier()

        def chunk(i, carry):
            base = w * EC + i * K
            pltpu.sync_copy(src_hbm.at[pl.ds(base, K)], sidx)
            pltpu.sync_copy(dst_hbm.at[pl.ds(base, K)], didx)
            pltpu.async_copy(t_hbm.at[sidx], buf, sem).wait()
            pltpu.sync_copy(buf, acc_sh.at[didx], add=True)
            return carry

        lax.fori_loop(0, NCHUNK, chunk, 0)
        plsc.subcore_barrier()
        pltpu.sync_copy(acc_sh.at[pl.ds(s * VROWS, VROWS)],
                        out_hbm.at[c, pl.ds(s * VROWS, VROWS)])

    return scat(src, dst, table, zerosVD)


def _tc_lin1(x, W1):
    def body(x_ref, w_ref, a_ref, b_ref):
        xb = x_ref[...].astype(jnp.bfloat16)
        wb = w_ref[...].astype(jnp.bfloat16)
        a_ref[...] = jnp.dot(xb, wb[:D, :],
                             preferred_element_type=jnp.float32)
        b_ref[...] = jnp.dot(xb, wb[D:, :],
                             preferred_element_type=jnp.float32)

    nb = V // BR
    return pl.pallas_call(
        body,
        grid=(nb,),
        in_specs=[pl.BlockSpec((BR, D), lambda i: (i, 0)),
                  pl.BlockSpec((2 * D, D), lambda i: (0, 0))],
        out_specs=[pl.BlockSpec((BR, D), lambda i: (i, 0)),
                   pl.BlockSpec((BR, D), lambda i: (i, 0))],
        out_shape=[jax.ShapeDtypeStruct((V, D), jnp.float32),
                   jax.ShapeDtypeStruct((V, D), jnp.float32)],
    )(x, W1)


def _tc_prep(cnt16):
    def body(c_ref, d_ref, q_ref):
        cnt = c_ref[0, :, 0:1] + c_ref[1, :, 0:1]
        deg = cnt + 1.0
        q_ref[...] = 1.0 / deg
        d_ref[...] = lax.rsqrt(deg)

    return pl.pallas_call(
        body,
        grid=(HEAD_BLKS,),
        in_specs=[pl.BlockSpec((NC, BR, D), lambda i: (0, i, 0))],
        out_specs=[pl.BlockSpec((BR, 1), lambda i: (i, 0)),
                   pl.BlockSpec((BR, 1), lambda i: (i, 0))],
        out_shape=[jax.ShapeDtypeStruct((V, 1), jnp.float32),
                   jax.ShapeDtypeStruct((V, 1), jnp.float32)],
    )(cnt16)


def _tc_scale_head(hlin, d_col):
    def body(h_ref, d_ref, o_ref):
        o_ref[...] = h_ref[...] * d_ref[...]

    return pl.pallas_call(
        body,
        grid=(HEAD_BLKS,),
        in_specs=[pl.BlockSpec((BR, D), lambda i: (i, 0)),
                  pl.BlockSpec((BR, 1), lambda i: (i, 0))],
        out_specs=pl.BlockSpec((BR, D), lambda i: (i, 0)),
        out_shape=jax.ShapeDtypeStruct((V, D), jnp.float32),
    )(hlin, d_col)


def _tc_head_affine(hlin, S1p, d_col, q_col, b1r):
    def body(h_ref, s_ref, d_ref, q_ref, b_ref, o_ref, sum_ref, ss_ref):
        i = pl.program_id(0)
        h1 = (h_ref[...] * q_ref[...]
              + d_ref[...] * (s_ref[0] + s_ref[1])
              + b_ref[...])
        o_ref[...] = h1

        @pl.when(i == 0)
        def _():
            sum_ref[...] = jnp.zeros_like(sum_ref)
            ss_ref[...] = jnp.zeros_like(ss_ref)

        sum_ref[...] += jnp.sum(h1, axis=0, keepdims=True)
        ss_ref[...] += jnp.sum(h1 * h1, axis=0, keepdims=True)

    return pl.pallas_call(
        body,
        grid=(HEAD_BLKS,),
        in_specs=[pl.BlockSpec((BR, D), lambda i: (i, 0)),
                  pl.BlockSpec((NC, BR, D), lambda i: (0, i, 0)),
                  pl.BlockSpec((BR, 1), lambda i: (i, 0)),
                  pl.BlockSpec((BR, 1), lambda i: (i, 0)),
                  pl.BlockSpec((1, D), lambda i: (0, 0))],
        out_specs=[pl.BlockSpec((BR, D), lambda i: (i, 0)),
                   pl.BlockSpec((1, D), lambda i: (0, 0)),
                   pl.BlockSpec((1, D), lambda i: (0, 0))],
        out_shape=[jax.ShapeDtypeStruct((V, D), jnp.float32),
                   jax.ShapeDtypeStruct((1, D), jnp.float32),
                   jax.ShapeDtypeStruct((1, D), jnp.float32)],
    )(hlin, S1p, d_col, q_col, b1r)


def _tc_tail_stats(hlin, b1r):
    def body(h_ref, b_ref, sum_ref, ss_ref):
        i = pl.program_id(0)
        t = h_ref[...] + b_ref[...]

        @pl.when(i == 0)
        def _():
            sum_ref[...] = jnp.zeros_like(sum_ref)
            ss_ref[...] = jnp.zeros_like(ss_ref)

        sum_ref[...] += jnp.sum(t, axis=0, keepdims=True)
        ss_ref[...] += jnp.sum(t * t, axis=0, keepdims=True)

    return pl.pallas_call(
        body,
        grid=(TAIL_BLKS,),
        in_specs=[pl.BlockSpec((BR, D), lambda i: (i + HEAD_BLKS, 0)),
                  pl.BlockSpec((1, D), lambda i: (0, 0))],
        out_specs=[pl.BlockSpec((1, D), lambda i: (0, 0)),
                   pl.BlockSpec((1, D), lambda i: (0, 0))],
        out_shape=[jax.ShapeDtypeStruct((1, D), jnp.float32),
                   jax.ShapeDtypeStruct((1, D), jnp.float32)],
    )(hlin, b1r)


def _tc_bn(sum_h, ss_h, sum_t, ss_t):
    def body(sh, qh, st, qt, mean_ref, sq_ref):
        mean = (sh[...] + st[...]) * (1.0 / E)
        ex2 = (qh[...] + qt[...]) * (1.0 / E)
        var = ex2 - mean * mean
        mean_ref[...] = mean
        sq_ref[...] = jnp.sqrt(var + 1e-5)

    return pl.pallas_call(
        body,
        out_shape=[jax.ShapeDtypeStruct((1, D), jnp.float32),
                   jax.ShapeDtypeStruct((1, D), jnp.float32)],
    )(sum_h, ss_h, sum_t, ss_t)


def _tc_head_mm(h1_head, mean, sq, gammar, betar, W2, d_col):
    def body(h_ref, m_ref, s_ref, g2_ref, be_ref, w_ref, d_ref, g_ref, t_ref):
        hn = (h_ref[...] - m_ref[...]) / s_ref[...] * g2_ref[...] + be_ref[...]
        r = jnp.maximum(hn, 0.0)
        g = jnp.dot(r.astype(jnp.bfloat16), w_ref[...].astype(jnp.bfloat16),
                    preferred_element_type=jnp.float32)
        g_ref[...] = g
        t_ref[...] = g * d_ref[...]

    return pl.pallas_call(
        body,
        grid=(HEAD_BLKS,),
        in_specs=[pl.BlockSpec((BR, D), lambda i: (i, 0)),
                  pl.BlockSpec((1, D), lambda i: (0, 0)),
                  pl.BlockSpec((1, D), lambda i: (0, 0)),
                  pl.BlockSpec((1, D), lambda i: (0, 0)),
                  pl.BlockSpec((1, D), lambda i: (0, 0)),
                  pl.BlockSpec((D, D), lambda i: (0, 0)),
                  pl.BlockSpec((BR, 1), lambda i: (i, 0))],
        out_specs=[pl.BlockSpec((BR, D), lambda i: (i, 0)),
                   pl.BlockSpec((BR, D), lambda i: (i, 0))],
        out_shape=[jax.ShapeDtypeStruct((V, D), jnp.float32),
                   jax.ShapeDtypeStruct((V, D), jnp.float32)],
    )(h1_head, mean, sq, gammar, betar, W2, d_col)


def _tc_tail_mm(hlin, b1r, mean, sq, gammar, betar, W2, b2r, Wl, blr, cut):
    def body(h_ref, b_ref, m_ref, s2_ref, g2_ref, be_ref, w2_ref, b2_ref,
             wl_ref, bl_ref, cut_ref, s_ref, rd_ref):
        h1 = h_ref[...] + b_ref[...]
        hn = (h1 - m_ref[...]) / s2_ref[...] * g2_ref[...] + be_ref[...]
        r = jnp.maximum(hn, 0.0)
        q = jnp.dot(r.astype(jnp.bfloat16), w2_ref[...].astype(jnp.bfloat16),
                    preferred_element_type=jnp.float32)
        h2 = q + b2_ref[...]
        h3 = jnp.dot(h2.astype(jnp.bfloat16),
                     wl_ref[...].astype(jnp.bfloat16),
                     preferred_element_type=jnp.float32) + bl_ref[...]
        sv = jax.nn.sigmoid(h3)
        s_ref[...] = sv
        rd_ref[...] = jnp.where(sv < cut_ref[0, 0], 0.0, 1.0)

    return pl.pallas_call(
        body,
        grid=(TAIL_BLKS,),
        in_specs=[pl.BlockSpec((BR, D), lambda i: (i + HEAD_BLKS, 0)),
                  pl.BlockSpec((1, D), lambda i: (0, 0)),
                  pl.BlockSpec((1, D), lambda i: (0, 0)),
                  pl.BlockSpec((1, D), lambda i: (0, 0)),
                  pl.BlockSpec((1, D), lambda i: (0, 0)),
                  pl.BlockSpec((1, D), lambda i: (0, 0)),
                  pl.BlockSpec((D, D), lambda i: (0, 0)),
                  pl.BlockSpec((1, D), lambda i: (0, 0)),
                  pl.BlockSpec((D, D), lambda i: (0, 0)),
                  pl.BlockSpec((1, D), lambda i: (0, 0)),
                  pl.BlockSpec((1, 1), lambda i: (0, 0))],
        out_specs=[pl.BlockSpec((BR, D), lambda i: (i + HEAD_BLKS, 0)),
                   pl.BlockSpec((BR, D), lambda i: (i + HEAD_BLKS, 0))],
        out_shape=[jax.ShapeDtypeStruct((E, D), jnp.float32),
                   jax.ShapeDtypeStruct((E, D), jnp.float32)],
    )(hlin, b1r, mean, sq, gammar, betar, W2, b2r, Wl, blr, cut)


def _tc_head_final(g_head, S2p, d_col, q_col, b2r, Wl, blr, cut,
                   s_full, rd_full):
    def body(g_ref, s2_ref, d_ref, q_ref, b2_ref, wl_ref, bl_ref, cut_ref,
             sin_ref, rin_ref, s_ref, rd_ref):
        h2 = (g_ref[...] * q_ref[...]
              + d_ref[...] * (s2_ref[0] + s2_ref[1])
              + b2_ref[...])
        h3 = jnp.dot(h2.astype(jnp.bfloat16),
                     wl_ref[...].astype(jnp.bfloat16),
                     preferred_element_type=jnp.float32) + bl_ref[...]
        sv = jax.nn.sigmoid(h3)
        s_ref[...] = sv
        rd_ref[...] = jnp.where(sv < cut_ref[0, 0], 0.0, 1.0)

    return pl.pallas_call(
        body,
        grid=(HEAD_BLKS,),
        in_specs=[pl.BlockSpec((BR, D), lambda i: (i, 0)),
                  pl.BlockSpec((NC, BR, D), lambda i: (0, i, 0)),
                  pl.BlockSpec((BR, 1), lambda i: (i, 0)),
                  pl.BlockSpec((BR, 1), lambda i: (i, 0)),
                  pl.BlockSpec((1, D), lambda i: (0, 0)),
                  pl.BlockSpec((D, D), lambda i: (0, 0)),
                  pl.BlockSpec((1, D), lambda i: (0, 0)),
                  pl.BlockSpec((1, 1), lambda i: (0, 0)),
                  pl.BlockSpec((BR, D), lambda i: (i, 0)),
                  pl.BlockSpec((BR, D), lambda i: (i, 0))],
        out_specs=[pl.BlockSpec((BR, D), lambda i: (i, 0)),
                   pl.BlockSpec((BR, D), lambda i: (i, 0))],
        out_shape=[jax.ShapeDtypeStruct((E, D), jnp.float32),
                   jax.ShapeDtypeStruct((E, D), jnp.float32)],
        input_output_aliases={8: 0, 9: 1},
    )(g_head, S2p, d_col, q_col, b2r, Wl, blr, cut, s_full, rd_full)


def kernel(x, edge_index, cutoff, W1, b1, gamma, beta, W2, b2, Wl, bl):
    src = edge_index[0]
    dst = edge_index[1]
    b1r = b1.reshape(1, D)
    b2r = b2.reshape(1, D)
    blr = bl.reshape(1, D)
    gammar = gamma.reshape(1, D)
    betar = beta.reshape(1, D)
    cut = cutoff.reshape(1, 1)

    onesKD = jnp.ones((K, D), jnp.float32)
    zerosVD = jnp.zeros((V2, D), jnp.float32)

    A, B = _tc_lin1(x, W1)

    cnt16 = _sc_hist(dst, zerosVD, onesKD)
    d_col, q_col = _tc_prep(cnt16)

    hlin = _sc_gather_add(src, dst, A, B)

    T1s = _tc_scale_head(hlin, d_col)
    S1p = _sc_scatter(src, dst, T1s, zerosVD)

    h1_head, sum_h, ss_h = _tc_head_affine(hlin, S1p, d_col, q_col, b1r)
    sum_t, ss_t = _tc_tail_stats(hlin, b1r)
    mean, sq = _tc_bn(sum_h, ss_h, sum_t, ss_t)

    g_head, T2s = _tc_head_mm(h1_head, mean, sq, gammar, betar, W2, d_col)
    s_full, rd_full = _tc_tail_mm(hlin, b1r, mean, sq, gammar, betar, W2,
                                  b2r, Wl, blr, cut)

    S2p = _sc_scatter(src, dst, T2s, zerosVD)
    s_out, rd_out = _tc_head_final(g_head, S2p, d_col, q_col, b2r, Wl, blr,
                                   cut, s_full, rd_full)

    return s_out, lax.stop_gradient(rd_out)
